# baseline (speedup 1.0000x reference)
import jax
import jax.numpy as jnp
from jax import lax
from jax.experimental import pallas as pl
from jax.experimental.pallas import tpu as pltpu
from jax.experimental.pallas import tpu_sc as plsc

SEQ_LEN = 200
EMBED_DIM = 64
BATCH = 4096
VOCAB = 100000
PADW = 128

NC, NS, LANES = 2, 16, 16
NW = NC * NS
BSTRIPE = BATCH // NW


def _body(idxT_hbm, tok_hbm, pos_hbm, out_hbm, idx_v, rows_v, outb_v, pos_v,
          g_sem, o_sem):
    wid = lax.axis_index("s") * NC + lax.axis_index("c")
    b0 = wid * BSTRIPE
    pltpu.sync_copy(pos_hbm, pos_v)

    iota = lax.iota(jnp.int32, LANES)
    jvecs = [iota + j * LANES for j in range(BSTRIPE // LANES)]

    def start_gather(l, p):
        pltpu.sync_copy(idxT_hbm.at[l, pl.ds(b0, BSTRIPE)], idx_v.at[p])
        pltpu.async_copy(tok_hbm.at[idx_v.at[p]], rows_v.at[p], g_sem.at[p])

    def wait_gather(p):
        pltpu.make_async_copy(tok_hbm.at[idx_v.at[p]], rows_v.at[p],
                              g_sem.at[p]).wait()

    def start_out(l, p):
        pltpu.async_copy(outb_v.at[p], out_hbm.at[l, :, pl.ds(b0, BSTRIPE)],
                         o_sem.at[p])

    def wait_out(l, p):
        pltpu.make_async_copy(outb_v.at[p], out_hbm.at[l, :, pl.ds(b0, BSTRIPE)],
                              o_sem.at[p]).wait()

    start_gather(0, 0)
    start_gather(1, 1)

    def step(i, p):
        l = 2 * i + p
        wait_gather(p)
        lax.cond(i >= 1, lambda: wait_out(l - 2, p), lambda: None)
        lsplat = jnp.full((LANES,), 0, jnp.int32) + l

        @plsc.parallel_loop(0, EMBED_DIM, unroll=2)
        def transpose_add(d):
            dsplat = jnp.full((LANES,), 0, jnp.int32) + d
            pos16 = plsc.load_gather(pos_v, [lsplat, dsplat])
            datas = [plsc.load_gather(rows_v.at[p], [jvecs[j], dsplat])
                     for j in range(BSTRIPE // LANES)]
            for j in range(BSTRIPE // LANES):
                outb_v[p, d, pl.ds(j * LANES, LANES)] = datas[j] + pos16

        start_out(l, p)
        lax.cond(i < SEQ_LEN // 2 - 1, lambda: start_gather(l + 2, p),
                 lambda: None)

    def outer(i, carry):
        step(i, 0)
        step(i, 1)
        return carry

    lax.fori_loop(0, SEQ_LEN // 2, outer, 0)
    wait_out(SEQ_LEN - 2, 0)
    wait_out(SEQ_LEN - 1, 1)


_mesh = plsc.VectorSubcoreMesh(core_axis_name="c", subcore_axis_name="s")

_gather = pl.kernel(
    _body,
    out_type=jax.ShapeDtypeStruct((SEQ_LEN, EMBED_DIM, BATCH), jnp.float32),
    mesh=_mesh,
    scratch_types=[
        pltpu.VMEM((2, BSTRIPE), jnp.int32),
        pltpu.VMEM((2, BSTRIPE, PADW), jnp.float32),
        pltpu.VMEM((2, EMBED_DIM, BSTRIPE), jnp.float32),
        pltpu.VMEM((SEQ_LEN, PADW), jnp.float32),
        pltpu.SemaphoreType.DMA((2,)),
        pltpu.SemaphoreType.DMA((2,)),
    ],
    compiler_params=pltpu.CompilerParams(use_tc_tiling_on_sc=True,
                                         needs_layout_passes=False),
)


@jax.jit
def kernel(inputs, token_table, position_table):
    idxT = inputs.astype(jnp.int32).T
    tok_pad = jnp.pad(token_table, ((0, 0), (0, PADW - EMBED_DIM)))
    pos_pad = jnp.pad(position_table, ((0, 0), (0, PADW - EMBED_DIM)))
    out_t = _gather(idxT, tok_pad, pos_pad)
    return out_t.transpose(2, 0, 1)


# speedup vs baseline: 1.0896x; 1.0896x over previous
import jax
import jax.numpy as jnp
from jax import lax
from jax.experimental import pallas as pl
from jax.experimental.pallas import tpu as pltpu
from jax.experimental.pallas import tpu_sc as plsc

SEQ_LEN = 200
EMBED_DIM = 64
BATCH = 4096
VOCAB = 100000
PADW = 128
PITCH = 136

NC, NS, LANES = 2, 16, 16
NW = NC * NS
BSTRIPE = BATCH // NW
NJ = BSTRIPE // LANES


def _body(idxT_hbm, tok_hbm, possp_hbm, out_hbm, idx_v, rows_v, outb_v, pos_v,
          g_sem, o_sem):
    wid = lax.axis_index("s") * NC + lax.axis_index("c")
    b0 = wid * BSTRIPE

    iota = lax.iota(jnp.int32, LANES)
    jvecs = [iota + j * LANES for j in range(NJ)]

    def start_gather(l, p):
        pltpu.sync_copy(idxT_hbm.at[l, pl.ds(b0, BSTRIPE)], idx_v.at[p])
        pltpu.sync_copy(possp_hbm.at[l], pos_v.at[p])
        pltpu.async_copy(tok_hbm.at[idx_v.at[p]],
                         rows_v.at[p, :, pl.ds(0, PADW)], g_sem.at[p])

    def wait_gather(p):
        pltpu.make_async_copy(tok_hbm.at[idx_v.at[p]],
                              rows_v.at[p, :, pl.ds(0, PADW)],
                              g_sem.at[p]).wait()

    def start_out(l, p):
        pltpu.async_copy(outb_v.at[p], out_hbm.at[l, :, pl.ds(b0, BSTRIPE)],
                         o_sem.at[p])

    def wait_out(l, p):
        pltpu.make_async_copy(outb_v.at[p], out_hbm.at[l, :, pl.ds(b0, BSTRIPE)],
                              o_sem.at[p]).wait()

    start_gather(0, 0)
    start_gather(1, 1)

    def step(i, p):
        l = 2 * i + p
        wait_gather(p)
        lax.cond(i >= 1, lambda: wait_out(l - 2, p), lambda: None)

        @plsc.parallel_loop(0, EMBED_DIM, unroll=2)
        def transpose_add(d):
            dsplat = jnp.full((LANES,), 0, jnp.int32) + d
            pos16 = pos_v[p, pl.ds(d * LANES, LANES)]
            for j in range(NJ):
                data = plsc.load_gather(rows_v.at[p], [jvecs[j], dsplat])
                outb_v[p, d, pl.ds(j * LANES, LANES)] = data + pos16

        start_out(l, p)
        lax.cond(i < SEQ_LEN // 2 - 1, lambda: start_gather(l + 2, p),
                 lambda: None)

    def outer(i, carry):
        step(i, 0)
        step(i, 1)
        return carry

    lax.fori_loop(0, SEQ_LEN // 2, outer, 0)
    wait_out(SEQ_LEN - 2, 0)
    wait_out(SEQ_LEN - 1, 1)


_mesh = plsc.VectorSubcoreMesh(core_axis_name="c", subcore_axis_name="s")

_gather = pl.kernel(
    _body,
    out_type=jax.ShapeDtypeStruct((SEQ_LEN, EMBED_DIM, BATCH), jnp.float32),
    mesh=_mesh,
    scratch_types=[
        pltpu.VMEM((2, BSTRIPE), jnp.int32),
        pltpu.VMEM((2, BSTRIPE, PITCH), jnp.float32),
        pltpu.VMEM((2, EMBED_DIM, BSTRIPE), jnp.float32),
        pltpu.VMEM((2, EMBED_DIM * LANES), jnp.float32),
        pltpu.SemaphoreType.DMA((2,)),
        pltpu.SemaphoreType.DMA((2,)),
    ],
    compiler_params=pltpu.CompilerParams(use_tc_tiling_on_sc=True,
                                         needs_layout_passes=False),
)


@jax.jit
def kernel(inputs, token_table, position_table):
    idxT = inputs.astype(jnp.int32).T
    tok_pad = jnp.pad(token_table, ((0, 0), (0, PADW - EMBED_DIM)))
    pos_splat = jnp.broadcast_to(position_table[:, :, None],
                                 (SEQ_LEN, EMBED_DIM, LANES))
    pos_splat = pos_splat.reshape(SEQ_LEN, EMBED_DIM * LANES)
    out_t = _gather(idxT, tok_pad, pos_splat)
    return out_t.transpose(2, 0, 1)


# speedup vs baseline: 2.3036x; 2.1141x over previous
import jax
import jax.numpy as jnp
from jax import lax
from jax.experimental import pallas as pl
from jax.experimental.pallas import tpu as pltpu
from jax.experimental.pallas import tpu_sc as plsc

SEQ_LEN = 200
EMBED_DIM = 64
BATCH = 4096
VOCAB = 100000
PADW = 128

NC, NS, LANES = 2, 16, 16
NW = NC * NS
BSTRIPE = BATCH // NW
NJ = BSTRIPE // LANES
ND = EMBED_DIM // LANES


def _body(idxT_hbm, tok_hbm, pos_hbm, out_hbm, idx_v, rows_v, outb_v, pos_v,
          g_sem, o_sem):
    wid = lax.axis_index("s") * NC + lax.axis_index("c")
    b0 = wid * BSTRIPE
    pltpu.sync_copy(pos_hbm, pos_v)

    iota = lax.iota(jnp.int32, LANES)
    jvecs = [iota + j * LANES for j in range(NJ)]

    def start_gather(l, p):
        pltpu.sync_copy(idxT_hbm.at[l, pl.ds(b0, BSTRIPE)], idx_v.at[p])
        pltpu.async_copy(tok_hbm.at[idx_v.at[p]], rows_v.at[p], g_sem.at[p])

    def wait_gather(p):
        pltpu.make_async_copy(tok_hbm.at[idx_v.at[p]], rows_v.at[p],
                              g_sem.at[p]).wait()

    def start_out(l, p):
        pltpu.async_copy(outb_v.at[p], out_hbm.at[l, :, pl.ds(b0, BSTRIPE)],
                         o_sem.at[p])

    def wait_out(l, p):
        pltpu.make_async_copy(outb_v.at[p], out_hbm.at[l, :, pl.ds(b0, BSTRIPE)],
                              o_sem.at[p]).wait()

    start_gather(0, 0)
    start_gather(1, 1)

    def step(i, p):
        l = 2 * i + p
        wait_gather(p)
        lax.cond(i >= 1, lambda: wait_out(l - 2, p), lambda: None)
        lsplat = jnp.full((LANES,), 0, jnp.int32) + l

        for dg in range(ND):
            @plsc.parallel_loop(0, LANES, unroll=2)
            def diag_k(k):
                dvec = ((iota + k) & (LANES - 1)) + dg * LANES
                pos_diag = plsc.load_gather(pos_v, [lsplat, dvec])
                for j in range(NJ):
                    data = plsc.load_gather(rows_v.at[p], [jvecs[j], dvec])
                    plsc.store_scatter(outb_v.at[p], [dvec, jvecs[j]],
                                       data + pos_diag)

        start_out(l, p)
        lax.cond(i < SEQ_LEN // 2 - 1, lambda: start_gather(l + 2, p),
                 lambda: None)

    def outer(i, carry):
        step(i, 0)
        step(i, 1)
        return carry

    lax.fori_loop(0, SEQ_LEN // 2, outer, 0)
    wait_out(SEQ_LEN - 2, 0)
    wait_out(SEQ_LEN - 1, 1)


_mesh = plsc.VectorSubcoreMesh(core_axis_name="c", subcore_axis_name="s")

_gather = pl.kernel(
    _body,
    out_type=jax.ShapeDtypeStruct((SEQ_LEN, EMBED_DIM, BATCH), jnp.float32),
    mesh=_mesh,
    scratch_types=[
        pltpu.VMEM((2, BSTRIPE), jnp.int32),
        pltpu.VMEM((2, BSTRIPE, PADW), jnp.float32),
        pltpu.VMEM((2, EMBED_DIM, BSTRIPE), jnp.float32),
        pltpu.VMEM((SEQ_LEN, EMBED_DIM), jnp.float32),
        pltpu.SemaphoreType.DMA((2,)),
        pltpu.SemaphoreType.DMA((2,)),
    ],
    compiler_params=pltpu.CompilerParams(use_tc_tiling_on_sc=True,
                                         needs_layout_passes=False),
)


@jax.jit
def kernel(inputs, token_table, position_table):
    idxT = inputs.astype(jnp.int32).T
    tok_pad = jnp.pad(token_table, ((0, 0), (0, PADW - EMBED_DIM)))
    out_t = _gather(idxT, tok_pad, position_table)
    return out_t.transpose(2, 0, 1)


# speedup vs baseline: 2.3067x; 1.0013x over previous
import jax
import jax.numpy as jnp
from jax import lax
from jax.experimental import pallas as pl
from jax.experimental.pallas import tpu as pltpu
from jax.experimental.pallas import tpu_sc as plsc

SEQ_LEN = 200
EMBED_DIM = 64
BATCH = 4096
VOCAB = 100000
PADW = 128

NC, NS, LANES = 2, 16, 16
NW = NC * NS
BSTRIPE = BATCH // NW
NJ = BSTRIPE // LANES
ND = EMBED_DIM // LANES


def _body(idxT_hbm, tok_hbm, pos_hbm, out_hbm, idx_v, rows_v, outb_v, pos_v,
          g_sem, o_sem):
    wid = lax.axis_index("s") * NC + lax.axis_index("c")
    b0 = wid * BSTRIPE
    pltpu.sync_copy(pos_hbm, pos_v)

    iota = lax.iota(jnp.int32, LANES)
    jvecs = [iota + j * LANES for j in range(NJ)]

    def start_gather(l, p):
        pltpu.sync_copy(idxT_hbm.at[l, pl.ds(b0, BSTRIPE)], idx_v.at[p])
        pltpu.async_copy(tok_hbm.at[idx_v.at[p]], rows_v.at[p], g_sem.at[p])

    def wait_gather(p):
        pltpu.make_async_copy(tok_hbm.at[idx_v.at[p]], rows_v.at[p],
                              g_sem.at[p]).wait()

    def start_out(l, p):
        pltpu.async_copy(outb_v.at[p], out_hbm.at[l, :, pl.ds(b0, BSTRIPE)],
                         o_sem.at[p])

    def wait_out(l, p):
        pltpu.make_async_copy(outb_v.at[p], out_hbm.at[l, :, pl.ds(b0, BSTRIPE)],
                              o_sem.at[p]).wait()

    start_gather(0, 0)
    start_gather(1, 1)

    def step(i, p):
        l = 2 * i + p
        wait_gather(p)
        lax.cond(i >= 1, lambda: wait_out(l - 2, p), lambda: None)
        lsplat = jnp.full((LANES,), 0, jnp.int32) + l

        for dg in range(ND):
            @plsc.parallel_loop(0, LANES, unroll=4)
            def diag_k(k):
                dvec = ((iota + k) & (LANES - 1)) + dg * LANES
                pos_diag = plsc.load_gather(pos_v, [lsplat, dvec])
                for j in range(NJ):
                    data = plsc.load_gather(rows_v.at[p], [jvecs[j], dvec])
                    plsc.store_scatter(outb_v.at[p], [dvec, jvecs[j]],
                                       data + pos_diag)

        start_out(l, p)
        lax.cond(i < SEQ_LEN // 2 - 1, lambda: start_gather(l + 2, p),
                 lambda: None)

    def outer(i, carry):
        step(i, 0)
        step(i, 1)
        return carry

    lax.fori_loop(0, SEQ_LEN // 2, outer, 0)
    wait_out(SEQ_LEN - 2, 0)
    wait_out(SEQ_LEN - 1, 1)


_mesh = plsc.VectorSubcoreMesh(core_axis_name="c", subcore_axis_name="s")

_gather = pl.kernel(
    _body,
    out_type=jax.ShapeDtypeStruct((SEQ_LEN, EMBED_DIM, BATCH), jnp.float32),
    mesh=_mesh,
    scratch_types=[
        pltpu.VMEM((2, BSTRIPE), jnp.int32),
        pltpu.VMEM((2, BSTRIPE, PADW), jnp.float32),
        pltpu.VMEM((2, EMBED_DIM, BSTRIPE), jnp.float32),
        pltpu.VMEM((SEQ_LEN, EMBED_DIM), jnp.float32),
        pltpu.SemaphoreType.DMA((2,)),
        pltpu.SemaphoreType.DMA((2,)),
    ],
    compiler_params=pltpu.CompilerParams(use_tc_tiling_on_sc=True,
                                         needs_layout_passes=False),
)


@jax.jit
def kernel(inputs, token_table, position_table):
    idxT = inputs.astype(jnp.int32).T
    tok_pad = jnp.pad(token_table, ((0, 0), (0, PADW - EMBED_DIM)))
    out_t = _gather(idxT, tok_pad, position_table)
    return out_t.transpose(2, 0, 1)


# speedup vs baseline: 2.9487x; 1.2783x over previous
import jax
import jax.numpy as jnp
from jax import lax
from jax.experimental import pallas as pl
from jax.experimental.pallas import tpu as pltpu
from jax.experimental.pallas import tpu_sc as plsc

SEQ_LEN = 200
EMBED_DIM = 64
BATCH = 4096
VOCAB = 100000
PADW = 128

NC, NS, LANES = 2, 16, 16
NW = NC * NS
BSTRIPE = BATCH // NW
NJ = BSTRIPE // LANES
ND = EMBED_DIM // LANES


def _body(idxT_hbm, tok_hbm, pos_hbm, out_hbm, idx_v, rows_v, outb_v, pos_v,
          g_sem, o_sem):
    wid = lax.axis_index("s") * NC + lax.axis_index("c")
    b0 = wid * BSTRIPE
    pltpu.sync_copy(pos_hbm, pos_v)

    iota = lax.iota(jnp.int32, LANES)
    rvecs = [iota * NJ + c for c in range(NJ)]
    dstep = iota * 13

    def start_gather(l, p):
        pltpu.sync_copy(idxT_hbm.at[l, pl.ds(b0, BSTRIPE)], idx_v.at[p])
        pltpu.async_copy(tok_hbm.at[idx_v.at[p]], rows_v.at[p], g_sem.at[p])

    def wait_gather(p):
        pltpu.make_async_copy(tok_hbm.at[idx_v.at[p]], rows_v.at[p],
                              g_sem.at[p]).wait()

    def start_out(l, p):
        pltpu.async_copy(outb_v.at[p], out_hbm.at[l, :, pl.ds(b0, BSTRIPE)],
                         o_sem.at[p])

    def wait_out(l, p):
        pltpu.make_async_copy(outb_v.at[p], out_hbm.at[l, :, pl.ds(b0, BSTRIPE)],
                              o_sem.at[p]).wait()

    start_gather(0, 0)
    start_gather(1, 1)

    def step(i, p):
        l = 2 * i + p
        wait_gather(p)
        lax.cond(i >= 1, lambda: wait_out(l - 2, p), lambda: None)
        lsplat = jnp.full((LANES,), 0, jnp.int32) + l

        @plsc.parallel_loop(0, EMBED_DIM, unroll=4)
        def diag_k(k):
            dvec = (dstep + k) & (EMBED_DIM - 1)
            pos_diag = plsc.load_gather(pos_v, [lsplat, dvec])
            for c in range(NJ):
                data = plsc.load_gather(rows_v.at[p], [rvecs[c], dvec])
                plsc.store_scatter(outb_v.at[p], [dvec, rvecs[c]],
                                   data + pos_diag)

        start_out(l, p)
        lax.cond(i < SEQ_LEN // 2 - 1, lambda: start_gather(l + 2, p),
                 lambda: None)

    def outer(i, carry):
        step(i, 0)
        step(i, 1)
        return carry

    lax.fori_loop(0, SEQ_LEN // 2, outer, 0)
    wait_out(SEQ_LEN - 2, 0)
    wait_out(SEQ_LEN - 1, 1)


_mesh = plsc.VectorSubcoreMesh(core_axis_name="c", subcore_axis_name="s")

_gather = pl.kernel(
    _body,
    out_type=jax.ShapeDtypeStruct((SEQ_LEN, EMBED_DIM, BATCH), jnp.float32),
    mesh=_mesh,
    scratch_types=[
        pltpu.VMEM((2, BSTRIPE), jnp.int32),
        pltpu.VMEM((2, BSTRIPE, PADW), jnp.float32),
        pltpu.VMEM((2, EMBED_DIM, BSTRIPE), jnp.float32),
        pltpu.VMEM((SEQ_LEN, EMBED_DIM), jnp.float32),
        pltpu.SemaphoreType.DMA((2,)),
        pltpu.SemaphoreType.DMA((2,)),
    ],
    compiler_params=pltpu.CompilerParams(use_tc_tiling_on_sc=True,
                                         needs_layout_passes=False),
)


@jax.jit
def kernel(inputs, token_table, position_table):
    idxT = inputs.astype(jnp.int32).T
    tok_pad = jnp.pad(token_table, ((0, 0), (0, PADW - EMBED_DIM)))
    out_t = _gather(idxT, tok_pad, position_table)
    return out_t.transpose(2, 0, 1)
